# baseline (speedup 1.0000x reference)
import functools

import jax
import jax.numpy as jnp
from jax import lax
from jax.experimental import pallas as pl
from jax.experimental.pallas import tpu as pltpu
from jax.experimental.pallas import tpu_sc as plsc

VOCAB = 100000
EMBED = 64
B = 1024
L = 50

NW = 32
EX_PER_W = B // NW
IDX_PER_W = EX_PER_W * L
CHUNK = 80
NCHUNK = IDX_PER_W // CHUNK
LANES = 16

VBLK = 2048


def _sc_pool_body(ctx_hbm, emb_hbm, out_hbm, idx_v, rows_v, pooled_v, sem):
    wid = lax.axis_index("s") * 2 + lax.axis_index("c")
    pltpu.sync_copy(ctx_hbm.at[wid], idx_v)
    copies = [
        pltpu.async_copy(
            emb_hbm.at[idx_v.at[j]],
            rows_v.at[pl.ds(j * CHUNK, CHUNK), :],
            sem,
        )
        for j in range(NCHUNK)
    ]
    for cp in copies:
        cp.wait()

    def body(e, carry):
        base = e * L
        for c in range(EMBED // LANES):
            acc = rows_v[base, pl.ds(c * LANES, LANES)]
            for l in range(1, L):
                acc = acc + rows_v[base + l, pl.ds(c * LANES, LANES)]
            pooled_v[e, pl.ds(c * LANES, LANES)] = acc
        return carry

    lax.fori_loop(0, EX_PER_W, body, 0)
    pltpu.sync_copy(pooled_v, out_hbm.at[pl.ds(wid * EX_PER_W, EX_PER_W), :])


_sc_pool = functools.partial(
    pl.kernel,
    out_type=jax.ShapeDtypeStruct((B, EMBED), jnp.float32),
    mesh=plsc.VectorSubcoreMesh(core_axis_name="c", subcore_axis_name="s"),
    scratch_types=[
        pltpu.VMEM((NCHUNK, CHUNK), jnp.int32),
        pltpu.VMEM((IDX_PER_W, EMBED), jnp.float32),
        pltpu.VMEM((EX_PER_W, EMBED), jnp.float32),
        pltpu.SemaphoreType.DMA,
    ],
    compiler_params=pltpu.CompilerParams(use_tc_tiling_on_sc=False),
)(_sc_pool_body)


def _mm_body(c_ref, p_ref, w_ref, b_ref, o_ref):
    x = p_ref[:] / c_ref[:]
    o_ref[:] = (
        lax.dot_general(
            x, w_ref[:], (((1,), (1,)), ((), ())),
            preferred_element_type=jnp.float32,
        )
        + b_ref[:]
    )


def kernel(context_indices, actual_C, embedding, W, b):
    ctx = context_indices.reshape(NW, NCHUNK, CHUNK)
    pooled = _sc_pool(ctx, embedding)

    c2d = actual_C.astype(jnp.float32).reshape(B, 1)
    b2d = b.reshape(1, VOCAB)
    nv = pl.cdiv(VOCAB, VBLK)
    scores = pl.pallas_call(
        _mm_body,
        grid=(nv,),
        in_specs=[
            pl.BlockSpec((B, 1), lambda i: (0, 0)),
            pl.BlockSpec((B, EMBED), lambda i: (0, 0)),
            pl.BlockSpec((VBLK, EMBED), lambda i: (i, 0)),
            pl.BlockSpec((1, VBLK), lambda i: (0, i)),
        ],
        out_specs=pl.BlockSpec((B, VBLK), lambda i: (0, i)),
        out_shape=jax.ShapeDtypeStruct((B, VOCAB), jnp.float32),
    )(c2d, pooled, W, b2d)
    return scores


# speedup vs baseline: 1.0019x; 1.0019x over previous
import functools

import jax
import jax.numpy as jnp
from jax import lax
from jax.experimental import pallas as pl
from jax.experimental.pallas import tpu as pltpu
from jax.experimental.pallas import tpu_sc as plsc

VOCAB = 100000
EMBED = 64
B = 1024
L = 50

NW = 32
EX_PER_W = B // NW
IDX_PER_W = EX_PER_W * L
CHUNK = 80
NCHUNK = IDX_PER_W // CHUNK
LANES = 16

VBLK = 4096


def _sc_pool_body(ctx_hbm, emb_hbm, out_hbm, idx_v, rows_v, pooled_v, sem):
    wid = lax.axis_index("s") * 2 + lax.axis_index("c")
    pltpu.sync_copy(ctx_hbm.at[wid], idx_v)
    copies = [
        pltpu.async_copy(
            emb_hbm.at[idx_v.at[j]],
            rows_v.at[pl.ds(j * CHUNK, CHUNK), :],
            sem,
        )
        for j in range(NCHUNK)
    ]
    for cp in copies:
        cp.wait()

    def body(e, carry):
        base = e * L
        for c in range(EMBED // LANES):
            acc = rows_v[base, pl.ds(c * LANES, LANES)]
            for l in range(1, L):
                acc = acc + rows_v[base + l, pl.ds(c * LANES, LANES)]
            pooled_v[e, pl.ds(c * LANES, LANES)] = acc
        return carry

    lax.fori_loop(0, EX_PER_W, body, 0)
    pltpu.sync_copy(pooled_v, out_hbm.at[pl.ds(wid * EX_PER_W, EX_PER_W), :])


_sc_pool = functools.partial(
    pl.kernel,
    out_type=jax.ShapeDtypeStruct((B, EMBED), jnp.float32),
    mesh=plsc.VectorSubcoreMesh(core_axis_name="c", subcore_axis_name="s"),
    scratch_types=[
        pltpu.VMEM((NCHUNK, CHUNK), jnp.int32),
        pltpu.VMEM((IDX_PER_W, EMBED), jnp.float32),
        pltpu.VMEM((EX_PER_W, EMBED), jnp.float32),
        pltpu.SemaphoreType.DMA,
    ],
    compiler_params=pltpu.CompilerParams(use_tc_tiling_on_sc=False),
)(_sc_pool_body)


def _mm_body(c_ref, p_ref, w_ref, b_ref, o_ref):
    x = p_ref[:] / c_ref[:]
    o_ref[:] = (
        lax.dot_general(
            x, w_ref[:], (((1,), (1,)), ((), ())),
            preferred_element_type=jnp.float32,
        )
        + b_ref[:]
    )


def kernel(context_indices, actual_C, embedding, W, b):
    ctx = context_indices.reshape(NW, NCHUNK, CHUNK)
    pooled = _sc_pool(ctx, embedding)

    c2d = actual_C.astype(jnp.float32).reshape(B, 1)
    b2d = b.reshape(1, VOCAB)
    nv = pl.cdiv(VOCAB, VBLK)
    scores = pl.pallas_call(
        _mm_body,
        grid=(nv,),
        in_specs=[
            pl.BlockSpec((B, 1), lambda i: (0, 0)),
            pl.BlockSpec((B, EMBED), lambda i: (0, 0)),
            pl.BlockSpec((VBLK, EMBED), lambda i: (i, 0)),
            pl.BlockSpec((1, VBLK), lambda i: (0, i)),
        ],
        out_specs=pl.BlockSpec((B, VBLK), lambda i: (0, i)),
        out_shape=jax.ShapeDtypeStruct((B, VOCAB), jnp.float32),
        compiler_params=pltpu.CompilerParams(
            vmem_limit_bytes=100 * 1024 * 1024,
        ),
    )(c2d, pooled, W, b2d)
    return scores
